# Initial kernel scaffold; baseline (speedup 1.0000x reference)
import jax
import jax.numpy as jnp
from jax.experimental import pallas as pl


def kernel(x, sw_mean_weight, sw_var_weight, weight, bias):
    raise NotImplementedError("write your pallas kernel here")


# speedup vs baseline: 3.3765x; 3.3765x over previous
import jax
import jax.numpy as jnp
from jax.experimental import pallas as pl
from jax.experimental.pallas import tpu as pltpu

N, C = 16384, 2048
CPG = 16
T = 5
EPS = 1e-5
SW = 256
S = C // SW
BN1 = 1024
NB1 = N // 2 // BN1
BN3 = 1024
NB3 = N // 2 // BN3


def _stats_kernel(x_ref, sxx_ref, sx_ref):
    nb = pl.program_id(1)

    @pl.when(nb == 0)
    def _():
        sxx_ref[...] = jnp.zeros_like(sxx_ref)
        sx_ref[...] = jnp.zeros_like(sx_ref)

    for s in range(S):
        xs = x_ref[:, s * SW:(s + 1) * SW]
        sxx_ref[0, s] += jax.lax.dot_general(
            xs, xs, (((0,), (0,)), ((), ())),
            preferred_element_type=jnp.float32)
        sx_ref[0, s:s + 1, :] += jnp.sum(xs, axis=0, keepdims=True)


def _whiten_kernel(scal_ref, sxxp_ref, sxp_ref, w2_ref, b2_ref,
                   wfin_ref, beff_ref):
    a = scal_ref[0]
    b = scal_ref[1]
    vw0 = scal_ref[2]
    ii = jax.lax.broadcasted_iota(jnp.int32, (SW, SW), 0)
    jj = jax.lax.broadcasted_iota(jnp.int32, (SW, SW), 1)
    eye = (ii == jj).astype(jnp.float32)
    mask = ((ii // CPG) == (jj // CPG)).astype(jnp.float32)
    inv_n = jnp.float32(1.0 / N)

    for s in range(S):
        mu = (sxp_ref[0, s:s + 1, :] + sxp_ref[1, s:s + 1, :]) * inv_n
        exx = (sxxp_ref[0, s] + sxxp_ref[1, s]) * inv_n
        outer = jax.lax.dot_general(
            mu, mu, (((0,), (0,)), ((), ())),
            preferred_element_type=jnp.float32)
        cov = vw0 * (mask * (exx - outer)) + EPS * eye
        tr_elem = jnp.sum(cov * eye, axis=0, keepdims=True)
        group_tr = jnp.dot(tr_elem, mask,
                           preferred_element_type=jnp.float32)
        r = 1.0 / group_tr
        covN = cov * r
        P = eye
        for _ in range(T):
            P2 = jnp.dot(P, P, preferred_element_type=jnp.float32)
            P3 = jnp.dot(P2, P, preferred_element_type=jnp.float32)
            P = 1.5 * P - 0.5 * jnp.dot(P3, covN,
                                        preferred_element_type=jnp.float32)
        wm = P * jnp.sqrt(r)
        b0 = wm * w2_ref[s:s + 1, :]
        wfin_ref[s] = a * b0
        beff_ref[s:s + 1, :] = b2_ref[s:s + 1, :] - b * jnp.dot(
            mu, b0, preferred_element_type=jnp.float32)


def _apply_kernel(x_ref, w_ref, bias_ref, o_ref):
    for s in range(S):
        xs = x_ref[:, s * SW:(s + 1) * SW]
        o_ref[:, s * SW:(s + 1) * SW] = jnp.dot(
            xs, w_ref[s], preferred_element_type=jnp.float32
        ) + bias_ref[s:s + 1, :]


def kernel(x, sw_mean_weight, sw_var_weight, weight, bias):
    mw = jax.nn.softmax(sw_mean_weight)
    vw = jax.nn.softmax(sw_var_weight)
    scal = jnp.stack([1.0 - mw[1], mw[0], vw[0]]).astype(jnp.float32)

    sxxp, sxp = pl.pallas_call(
        _stats_kernel,
        grid=(2, NB1),
        in_specs=[pl.BlockSpec((BN1, C), lambda p, nb: (p * NB1 + nb, 0))],
        out_specs=[
            pl.BlockSpec((1, S, SW, SW), lambda p, nb: (p, 0, 0, 0)),
            pl.BlockSpec((1, S, SW), lambda p, nb: (p, 0, 0)),
        ],
        out_shape=[
            jax.ShapeDtypeStruct((2, S, SW, SW), jnp.float32),
            jax.ShapeDtypeStruct((2, S, SW), jnp.float32),
        ],
        compiler_params=pltpu.CompilerParams(
            dimension_semantics=("parallel", "arbitrary"),
            vmem_limit_bytes=48 * 1024 * 1024,
        ),
        name="sw_stats",
    )(x)

    w2 = weight.reshape(S, SW)
    b2 = bias.reshape(S, SW)
    wfin, beff = pl.pallas_call(
        _whiten_kernel,
        in_specs=[
            pl.BlockSpec(memory_space=pltpu.SMEM),
            pl.BlockSpec(memory_space=pltpu.VMEM),
            pl.BlockSpec(memory_space=pltpu.VMEM),
            pl.BlockSpec(memory_space=pltpu.VMEM),
            pl.BlockSpec(memory_space=pltpu.VMEM),
        ],
        out_specs=[
            pl.BlockSpec(memory_space=pltpu.VMEM),
            pl.BlockSpec(memory_space=pltpu.VMEM),
        ],
        out_shape=[
            jax.ShapeDtypeStruct((S, SW, SW), jnp.float32),
            jax.ShapeDtypeStruct((S, SW), jnp.float32),
        ],
        name="sw_whiten",
    )(scal, sxxp, sxp, w2, b2)

    y = pl.pallas_call(
        _apply_kernel,
        grid=(2, NB3),
        in_specs=[
            pl.BlockSpec((BN3, C), lambda p, nb: (p * NB3 + nb, 0)),
            pl.BlockSpec((S, SW, SW), lambda p, nb: (0, 0, 0)),
            pl.BlockSpec((S, SW), lambda p, nb: (0, 0)),
        ],
        out_specs=pl.BlockSpec((BN3, C), lambda p, nb: (p * NB3 + nb, 0)),
        out_shape=jax.ShapeDtypeStruct((N, C), jnp.float32),
        compiler_params=pltpu.CompilerParams(
            dimension_semantics=("parallel", "arbitrary"),
            vmem_limit_bytes=48 * 1024 * 1024,
        ),
        name="sw_apply",
    )(x, wfin, beff)
    return y


# speedup vs baseline: 3.6710x; 1.0872x over previous
import jax
import jax.numpy as jnp
from jax.experimental import pallas as pl
from jax.experimental.pallas import tpu as pltpu

N, C = 16384, 2048
CPG = 16
T = 5
EPS = 1e-5
SW = 256
S = C // SW
BN1 = 2048
NB1 = N // BN1
BN3 = 1024
NB3 = N // BN3


def _stats_whiten_kernel(scal_ref, x_ref, w2_ref, b2_ref,
                         wfin_ref, beff_ref, sxx_scr, sx_scr):
    nb = pl.program_id(0)

    @pl.when(nb == 0)
    def _():
        sxx_scr[...] = jnp.zeros_like(sxx_scr)
        sx_scr[...] = jnp.zeros_like(sx_scr)

    for s in range(S):
        xs = x_ref[:, s * SW:(s + 1) * SW]
        sxx_scr[s] += jax.lax.dot_general(
            xs, xs, (((0,), (0,)), ((), ())),
            preferred_element_type=jnp.float32)
        sx_scr[s] += jnp.sum(xs, axis=0, keepdims=True)

    @pl.when(nb == NB1 - 1)
    def _():
        a = scal_ref[0]
        b = scal_ref[1]
        vw0 = scal_ref[2]
        ii = jax.lax.broadcasted_iota(jnp.int32, (SW, SW), 0)
        jj = jax.lax.broadcasted_iota(jnp.int32, (SW, SW), 1)
        eye = (ii == jj).astype(jnp.float32)
        mask = ((ii // CPG) == (jj // CPG)).astype(jnp.float32)
        inv_n = jnp.float32(1.0 / N)

        for s in range(S):
            mu = sx_scr[s] * inv_n
            exx = sxx_scr[s] * inv_n
            outer = jax.lax.dot_general(
                mu, mu, (((0,), (0,)), ((), ())),
                preferred_element_type=jnp.float32)
            cov = vw0 * (mask * (exx - outer)) + EPS * eye
            tr_elem = jnp.sum(cov * eye, axis=0, keepdims=True)
            group_tr = jnp.dot(tr_elem, mask,
                               preferred_element_type=jnp.float32)
            r = 1.0 / group_tr
            covNm = cov * (-0.5 * r)
            P = 1.5 * eye + covNm
            for _ in range(T - 1):
                P2 = jnp.dot(P, P, preferred_element_type=jnp.float32)
                P3 = jnp.dot(P2, P, preferred_element_type=jnp.float32)
                P = 1.5 * P + jnp.dot(P3, covNm,
                                      preferred_element_type=jnp.float32)
            wm = P * jnp.sqrt(r)
            b0 = wm * w2_ref[s]
            wfin_ref[s] = a * b0
            beff_ref[s] = b2_ref[s] - b * jnp.dot(
                mu, b0, preferred_element_type=jnp.float32)


def _apply_kernel(x_ref, w_ref, bias_ref, o_ref):
    for s in range(S):
        xs = x_ref[:, s * SW:(s + 1) * SW]
        o_ref[:, s * SW:(s + 1) * SW] = jnp.dot(
            xs, w_ref[s], preferred_element_type=jnp.float32
        ) + bias_ref[s]


def kernel(x, sw_mean_weight, sw_var_weight, weight, bias):
    mw = jax.nn.softmax(sw_mean_weight)
    vw = jax.nn.softmax(sw_var_weight)
    scal = jnp.stack([1.0 - mw[1], mw[0], vw[0]]).astype(jnp.float32)
    w2 = weight.reshape(S, 1, SW)
    b2 = bias.reshape(S, 1, SW)

    wfin, beff = pl.pallas_call(
        _stats_whiten_kernel,
        grid=(NB1,),
        in_specs=[
            pl.BlockSpec(memory_space=pltpu.SMEM),
            pl.BlockSpec((BN1, C), lambda nb: (nb, 0)),
            pl.BlockSpec((S, 1, SW), lambda nb: (0, 0, 0)),
            pl.BlockSpec((S, 1, SW), lambda nb: (0, 0, 0)),
        ],
        out_specs=[
            pl.BlockSpec((S, SW, SW), lambda nb: (0, 0, 0)),
            pl.BlockSpec((S, 1, SW), lambda nb: (0, 0, 0)),
        ],
        out_shape=[
            jax.ShapeDtypeStruct((S, SW, SW), jnp.float32),
            jax.ShapeDtypeStruct((S, 1, SW), jnp.float32),
        ],
        scratch_shapes=[
            pltpu.VMEM((S, SW, SW), jnp.float32),
            pltpu.VMEM((S, 1, SW), jnp.float32),
        ],
        compiler_params=pltpu.CompilerParams(
            dimension_semantics=("arbitrary",),
            vmem_limit_bytes=48 * 1024 * 1024,
        ),
        name="sw_stats_whiten",
    )(scal, x, w2, b2)

    y = pl.pallas_call(
        _apply_kernel,
        grid=(NB3,),
        in_specs=[
            pl.BlockSpec((BN3, C), lambda nb: (nb, 0)),
            pl.BlockSpec((S, SW, SW), lambda nb: (0, 0, 0)),
            pl.BlockSpec((S, 1, SW), lambda nb: (0, 0, 0)),
        ],
        out_specs=pl.BlockSpec((BN3, C), lambda nb: (nb, 0)),
        out_shape=jax.ShapeDtypeStruct((N, C), jnp.float32),
        compiler_params=pltpu.CompilerParams(
            dimension_semantics=("parallel",),
            vmem_limit_bytes=48 * 1024 * 1024,
        ),
        name="sw_apply",
    )(x, wfin, beff)
    return y


# speedup vs baseline: 3.6932x; 1.0060x over previous
import jax
import jax.numpy as jnp
from jax.experimental import pallas as pl
from jax.experimental.pallas import tpu as pltpu

N, C = 16384, 2048
CPG = 16
T = 5
EPS = 1e-5
SW = 256
S = C // SW
BN = 1024
NB = N // BN


def _fused_kernel(scal_ref, x_ref, w2_ref, b2_ref, o_ref,
                  sxx_scr, sx_scr, wfin_scr, beff_scr):
    i = pl.program_id(0)

    @pl.when(i == 0)
    def _():
        sxx_scr[...] = jnp.zeros_like(sxx_scr)
        sx_scr[...] = jnp.zeros_like(sx_scr)

    @pl.when(i < NB)
    def _():
        for s in range(S):
            xs = x_ref[:, s * SW:(s + 1) * SW]
            sxx_scr[s] += jax.lax.dot_general(
                xs, xs, (((0,), (0,)), ((), ())),
                preferred_element_type=jnp.float32)
            sx_scr[s] += jnp.sum(xs, axis=0, keepdims=True)

    @pl.when(i == NB - 1)
    def _():
        a = scal_ref[0]
        b = scal_ref[1]
        vw0 = scal_ref[2]
        ii = jax.lax.broadcasted_iota(jnp.int32, (SW, SW), 0)
        jj = jax.lax.broadcasted_iota(jnp.int32, (SW, SW), 1)
        eye = (ii == jj).astype(jnp.float32)
        mask = ((ii // CPG) == (jj // CPG)).astype(jnp.float32)
        inv_n = jnp.float32(1.0 / N)

        for s in range(S):
            mu = sx_scr[s] * inv_n
            exx = sxx_scr[s] * inv_n
            outer = jax.lax.dot_general(
                mu, mu, (((0,), (0,)), ((), ())),
                preferred_element_type=jnp.float32)
            cov = vw0 * (mask * (exx - outer)) + EPS * eye
            tr_elem = jnp.sum(cov * eye, axis=0, keepdims=True)
            group_tr = jnp.dot(tr_elem, mask,
                               preferred_element_type=jnp.float32)
            r = 1.0 / group_tr
            covNm = cov * (-0.5 * r)
            P = 1.5 * eye + covNm
            for _ in range(T - 1):
                P2 = jnp.dot(P, P, preferred_element_type=jnp.float32)
                P3 = jnp.dot(P2, P, preferred_element_type=jnp.float32)
                P = 1.5 * P + jnp.dot(P3, covNm,
                                      preferred_element_type=jnp.float32)
            wm = P * jnp.sqrt(r)
            b0 = wm * w2_ref[s]
            wfin_scr[s] = a * b0
            beff_scr[s] = b2_ref[s] - b * jnp.dot(
                mu, b0, preferred_element_type=jnp.float32)

    @pl.when(i >= NB)
    def _():
        for s in range(S):
            xs = x_ref[:, s * SW:(s + 1) * SW]
            o_ref[:, s * SW:(s + 1) * SW] = jnp.dot(
                xs, wfin_scr[s], preferred_element_type=jnp.float32
            ) + beff_scr[s]


def kernel(x, sw_mean_weight, sw_var_weight, weight, bias):
    mw = jax.nn.softmax(sw_mean_weight)
    vw = jax.nn.softmax(sw_var_weight)
    scal = jnp.stack([1.0 - mw[1], mw[0], vw[0]]).astype(jnp.float32)
    w2 = weight.reshape(S, 1, SW)
    b2 = bias.reshape(S, 1, SW)

    y = pl.pallas_call(
        _fused_kernel,
        grid=(2 * NB,),
        in_specs=[
            pl.BlockSpec(memory_space=pltpu.SMEM),
            pl.BlockSpec((BN, C), lambda i: (jax.lax.rem(i, NB), 0)),
            pl.BlockSpec((S, 1, SW), lambda i: (0, 0, 0)),
            pl.BlockSpec((S, 1, SW), lambda i: (0, 0, 0)),
        ],
        out_specs=pl.BlockSpec(
            (BN, C), lambda i: (jnp.where(i < NB, 0, i - NB), 0)),
        out_shape=jax.ShapeDtypeStruct((N, C), jnp.float32),
        scratch_shapes=[
            pltpu.VMEM((S, SW, SW), jnp.float32),
            pltpu.VMEM((S, 1, SW), jnp.float32),
            pltpu.VMEM((S, SW, SW), jnp.float32),
            pltpu.VMEM((S, 1, SW), jnp.float32),
        ],
        compiler_params=pltpu.CompilerParams(
            dimension_semantics=("arbitrary",),
            vmem_limit_bytes=48 * 1024 * 1024,
        ),
        name="sw_fused",
    )(scal, x, w2, b2)
    return y
